# baseline (speedup 1.0000x reference)
import functools

import jax
import jax.numpy as jnp
from jax import lax
from jax.experimental import pallas as pl
from jax.experimental.pallas import tpu as pltpu
from jax.experimental.pallas import tpu_sc as plsc

_F32 = jnp.float32
_BF16 = jnp.bfloat16


def _silu(x):
    return x * jax.nn.sigmoid(x)


def _mm(a, b):
    return jnp.dot(a.astype(_BF16), b.astype(_BF16),
                   preferred_element_type=_F32)


def _kron8(w):
    return jnp.kron(jnp.eye(8, dtype=_F32), w)


def _sc_gather(table, idx, chunk):
    n_rows, d = table.shape
    e = idx.shape[0]
    nw = 32
    epw = e // nw
    nch = epw // chunk
    mesh = plsc.VectorSubcoreMesh(core_axis_name="c", subcore_axis_name="s")

    @functools.partial(
        pl.kernel,
        mesh=mesh,
        compiler_params=pltpu.CompilerParams(use_tc_tiling_on_sc=False,
                                             needs_layout_passes=False),
        out_type=jax.ShapeDtypeStruct((e, d), table.dtype),
        scratch_types=[
            pltpu.VMEM((chunk,), jnp.int32),
            pltpu.VMEM((chunk,), jnp.int32),
            pltpu.VMEM((chunk, d), table.dtype),
            pltpu.VMEM((chunk, d), table.dtype),
            pltpu.SemaphoreType.DMA,
            pltpu.SemaphoreType.DMA,
        ],
    )
    def k(table_hbm, idx_hbm, out_hbm, i0, i1, r0, r1, s0, s1):
        wid = lax.axis_index("s") * 2 + lax.axis_index("c")
        base0 = wid * epw
        ib, rb, sb = [i0, i1], [r0, r1], [s0, s1]
        pltpu.sync_copy(idx_hbm.at[pl.ds(base0, chunk)], i0)
        descs = [pltpu.async_copy(table_hbm.at[i0], r0, s0), None]
        for c in range(nch):
            cur, nxt = c % 2, (c + 1) % 2
            if c + 1 < nch:
                pltpu.sync_copy(
                    idx_hbm.at[pl.ds(base0 + (c + 1) * chunk, chunk)], ib[nxt])
                descs[nxt] = pltpu.async_copy(table_hbm.at[ib[nxt]], rb[nxt],
                                              sb[nxt])
            descs[cur].wait()
            pltpu.sync_copy(rb[cur], out_hbm.at[pl.ds(base0 + c * chunk,
                                                      chunk)])

    return k(table, idx)


def _sc_scatter_add(vals, idx, zinit, chunk):
    e = vals.shape[0]
    n_rows = zinit.shape[0]
    nw = 32
    epw = e // nw
    nch = epw // chunk
    rps = n_rows // 16
    mesh = plsc.VectorSubcoreMesh(core_axis_name="c", subcore_axis_name="s")

    @functools.partial(
        pl.kernel,
        mesh=mesh,
        compiler_params=pltpu.CompilerParams(use_tc_tiling_on_sc=False,
                                             needs_layout_passes=False),
        out_type=jax.ShapeDtypeStruct((2, n_rows, 16), _F32),
        scratch_types=[
            pltpu.VMEM((chunk,), jnp.int32),
            pltpu.VMEM((chunk,), jnp.int32),
            pltpu.VMEM((chunk, 16), _F32),
            pltpu.VMEM((chunk, 16), _F32),
            pltpu.SemaphoreType.DMA,
            pltpu.SemaphoreType.DMA,
            pltpu.VMEM_SHARED((n_rows, 16), _F32),
        ],
    )
    def k(vals_hbm, idx_hbm, zinit_hbm, out_hbm, i0, i1, v0, v1, s0, s1,
          shared):
        cid = lax.axis_index("c")
        sid = lax.axis_index("s")
        wid = sid * 2 + cid
        base0 = wid * epw
        ib, vb, sb = [i0, i1], [v0, v1], [s0, s1]
        pltpu.sync_copy(zinit_hbm.at[pl.ds(sid * rps, rps)],
                        v0.at[pl.ds(0, rps)])
        pltpu.sync_copy(v0.at[pl.ds(0, rps)],
                        shared.at[pl.ds(sid * rps, rps)])
        plsc.subcore_barrier()

        pltpu.sync_copy(idx_hbm.at[pl.ds(base0, chunk)], i0)
        descs = [pltpu.async_copy(vals_hbm.at[pl.ds(base0, chunk)], v0, s0),
                 None]
        for c in range(nch):
            cur, nxt = c % 2, (c + 1) % 2
            if c + 1 < nch:
                base_n = base0 + (c + 1) * chunk
                pltpu.sync_copy(idx_hbm.at[pl.ds(base_n, chunk)], ib[nxt])
                descs[nxt] = pltpu.async_copy(
                    vals_hbm.at[pl.ds(base_n, chunk)], vb[nxt], sb[nxt])
            descs[cur].wait()
            pltpu.sync_copy(vb[cur], shared.at[ib[cur]], add=True)

        plsc.subcore_barrier()
        pltpu.sync_copy(shared.at[pl.ds(sid * rps, rps)],
                        v0.at[pl.ds(0, rps)])
        pltpu.sync_copy(v0.at[pl.ds(0, rps)],
                        out_hbm.at[cid, pl.ds(sid * rps, rps)])

    return k(vals, idx, zinit)


def _tc_init(atom_fea, edge_attr_p, W_atom, b_atom, K8We, b_edge8, K8ones,
             Ws0, Wd0):
    n, _ = atom_fea.shape
    e8 = edge_attr_p.shape[0]
    bn = 400
    b8 = bn * 4
    grid = n // bn

    def body(af, ea, wa, ba, we, beb, ko, ws, wd, node_o, ps_o, pd_o, edge_o,
             mf_o):
        nd = _mm(af[...], wa[...]) + ba[...]
        node_o[...] = nd
        ps_o[...] = _mm(nd, ws[...])
        pd_o[...] = _mm(nd, wd[...])
        ea_v = ea[...]
        gsum = _mm(jnp.abs(ea_v), ko[...])
        mf = (gsum > 1e-06).astype(_F32)
        mf_o[...] = mf
        edge_o[...] = (_mm(ea_v, we[...]) + beb[...]) * mf

    return pl.pallas_call(
        body,
        grid=(grid,),
        in_specs=[
            pl.BlockSpec((bn, 128), lambda i: (i, 0)),
            pl.BlockSpec((b8, 128), lambda i: (i, 0)),
            pl.BlockSpec((128, 64), lambda i: (0, 0)),
            pl.BlockSpec((1, 64), lambda i: (0, 0)),
            pl.BlockSpec((128, 128), lambda i: (0, 0)),
            pl.BlockSpec((1, 128), lambda i: (0, 0)),
            pl.BlockSpec((128, 128), lambda i: (0, 0)),
            pl.BlockSpec((64, 64), lambda i: (0, 0)),
            pl.BlockSpec((64, 64), lambda i: (0, 0)),
        ],
        out_specs=[
            pl.BlockSpec((bn, 64), lambda i: (i, 0)),
            pl.BlockSpec((bn, 64), lambda i: (i, 0)),
            pl.BlockSpec((bn, 64), lambda i: (i, 0)),
            pl.BlockSpec((b8, 128), lambda i: (i, 0)),
            pl.BlockSpec((b8, 128), lambda i: (i, 0)),
        ],
        out_shape=[
            jax.ShapeDtypeStruct((n, 64), _F32),
            jax.ShapeDtypeStruct((n, 64), _F32),
            jax.ShapeDtypeStruct((n, 64), _F32),
            jax.ShapeDtypeStruct((e8, 128), _F32),
            jax.ShapeDtypeStruct((e8, 128), _F32),
        ],
    )(atom_fea, edge_attr_p, W_atom, b_atom, K8We, b_edge8, K8ones, Ws0, Wd0)


def _tc_edge(edge_p, G2, ps, mask_p, K8U, b18, K8W2, b28):
    e8 = edge_p.shape[0]
    n = ps.shape[0]
    bn = 1000
    b8 = bn * 4
    b2 = bn * 16
    grid = n // bn

    def body(e_ref, g_ref, ps_ref, mf_ref, u_ref, b1_ref, w2_ref, b2_ref,
             eo_ref):
        psl = jnp.tile(ps_ref[...], (1, 8))
        psb = jnp.broadcast_to(psl[:, None, :], (bn, 4, 512))
        psr = psb.reshape(b8, 512)
        g8 = g_ref[...].reshape(b8, 512)
        ev = e_ref[...]
        pre = _mm(ev, u_ref[...]) + psr + g8 + b1_ref[...]
        h = _silu(pre)
        eo_ref[...] = (ev + _mm(h, w2_ref[...]) + b2_ref[...]) * mf_ref[...]

    return pl.pallas_call(
        body,
        grid=(grid,),
        in_specs=[
            pl.BlockSpec((b8, 128), lambda i: (i, 0)),
            pl.BlockSpec((b2, 128), lambda i: (i, 0)),
            pl.BlockSpec((bn, 64), lambda i: (i, 0)),
            pl.BlockSpec((b8, 128), lambda i: (i, 0)),
            pl.BlockSpec((128, 512), lambda i: (0, 0)),
            pl.BlockSpec((1, 512), lambda i: (0, 0)),
            pl.BlockSpec((512, 128), lambda i: (0, 0)),
            pl.BlockSpec((1, 128), lambda i: (0, 0)),
        ],
        out_specs=pl.BlockSpec((b8, 128), lambda i: (i, 0)),
        out_shape=jax.ShapeDtypeStruct((e8, 128), _F32),
    )(edge_p, G2, ps, mask_p, K8U, b18, K8W2, b28)


def _tc_node(node, aggP, aggP2, rinv, Wn1a, Wn1b, bn1, Wn2, bn2, Ws, Wd):
    n = node.shape[0]
    bn = n
    grid = n // bn

    def body(nd_ref, ag_ref, ag2_ref, ri_ref, w1a, w1b, b1r, w2r, b2r, wsr,
             wdr, no_ref, ps_ref, pd_ref):
        agv = ag_ref[...]
        ag2v = ag2_ref[...]
        agg = (agv[0] + agv[1] + ag2v[0] + ag2v[1]) * ri_ref[...]
        nd = nd_ref[...]
        h = _silu(_mm(nd, w1a[...]) + _mm(agg, w1b[...]) + b1r[...])
        nn = nd + _mm(h, w2r[...]) + b2r[...]
        no_ref[...] = nn
        ps_ref[...] = _mm(nn, wsr[...])
        pd_ref[...] = _mm(nn, wdr[...])

    return pl.pallas_call(
        body,
        grid=(grid,),
        in_specs=[
            pl.BlockSpec((bn, 64), lambda i: (i, 0)),
            pl.BlockSpec((2, bn, 16), lambda i: (0, i, 0)),
            pl.BlockSpec((2, bn, 16), lambda i: (0, i, 0)),
            pl.BlockSpec((bn, 16), lambda i: (i, 0)),
            pl.BlockSpec((64, 64), lambda i: (0, 0)),
            pl.BlockSpec((16, 64), lambda i: (0, 0)),
            pl.BlockSpec((1, 64), lambda i: (0, 0)),
            pl.BlockSpec((64, 64), lambda i: (0, 0)),
            pl.BlockSpec((1, 64), lambda i: (0, 0)),
            pl.BlockSpec((64, 64), lambda i: (0, 0)),
            pl.BlockSpec((64, 64), lambda i: (0, 0)),
        ],
        out_specs=[
            pl.BlockSpec((bn, 64), lambda i: (i, 0)),
            pl.BlockSpec((bn, 64), lambda i: (i, 0)),
            pl.BlockSpec((bn, 64), lambda i: (i, 0)),
        ],
        out_shape=[
            jax.ShapeDtypeStruct((n, 64), _F32),
            jax.ShapeDtypeStruct((n, 64), _F32),
            jax.ShapeDtypeStruct((n, 64), _F32),
        ],
    )(node, aggP, aggP2, rinv, Wn1a, Wn1b, bn1, Wn2, bn2, Ws, Wd)


def _tc_rinv(cntP):
    _, n, _ = cntP.shape
    bn = 2000
    grid = n // bn

    def body(c_ref, o_ref):
        cv = c_ref[...]
        cnt = cv[0] + cv[1]
        o_ref[...] = 1.0 / jnp.maximum(cnt, 1.0)

    return pl.pallas_call(
        body,
        grid=(grid,),
        in_specs=[pl.BlockSpec((2, bn, 16), lambda i: (0, i, 0))],
        out_specs=pl.BlockSpec((bn, 16), lambda i: (i, 0)),
        out_shape=jax.ShapeDtypeStruct((n, 16), _F32),
    )(cntP)


def _tc_readout(R, A, W_r, b_r, W_o, b_o):
    b = A.shape[0]
    ep = R.shape[0]

    def body(r_ref, a_ref, wr, br, wo, bo, o_ref):
        crys = _mm(a_ref[...], r_ref[...])
        cr = _silu(_mm(crys, wr[...]) + br[...])
        o_ref[...] = _mm(cr, wo[...]) + bo[...]

    return pl.pallas_call(
        body,
        grid=(1,),
        in_specs=[
            pl.BlockSpec((ep, 64), lambda i: (0, 0)),
            pl.BlockSpec((b, ep), lambda i: (0, 0)),
            pl.BlockSpec((64, 128), lambda i: (0, 0)),
            pl.BlockSpec((1, 128), lambda i: (0, 0)),
            pl.BlockSpec((128, 1), lambda i: (0, 0)),
            pl.BlockSpec((1, 1), lambda i: (0, 0)),
        ],
        out_specs=pl.BlockSpec((b, 1), lambda i: (0, 0)),
        out_shape=jax.ShapeDtypeStruct((b, 1), _F32),
    )(R, A, W_r, b_r, W_o, b_o)


def kernel(atom_fea, nbr_fea, nbr_fea_idx, crystal_atom_idx, W_atom, b_atom,
           W_edge, b_edge, W_e1, b_e1, W_e2, b_e2, W_n1, b_n1, W_n2, b_n2,
           W_r, b_r, W_o, b_o):
    n, m = nbr_fea_idx.shape
    e = n * m
    nl = W_e1.shape[0]
    b, p = crystal_atom_idx.shape

    edge_attr_p = nbr_fea.reshape(e // 8, 128)
    dst = jnp.clip(nbr_fea_idx.reshape(e), 0, n - 1).astype(jnp.int32)
    zinit = jnp.zeros((n, 16), _F32)

    node, ps, pd, edge_p, mask_p = _tc_init(
        atom_fea, edge_attr_p, W_atom, b_atom.reshape(1, 64),
        _kron8(W_edge), jnp.tile(b_edge, 8).reshape(1, 128),
        _kron8(jnp.ones((16, 16), _F32)),
        W_e1[0, 16:80], W_e1[0, 80:144])

    cntP = _sc_scatter_add(mask_p.reshape(e, 16), dst, zinit, 2000)
    rinv = _tc_rinv(cntP)

    eh = e // 2
    dst_a, dst_b = dst[:eh], dst[eh:]
    ea, eb = edge_p[:e // 16], edge_p[e // 16:]
    ma, mb = mask_p[:e // 16], mask_p[e // 16:]
    for l in range(nl):
        k8u = _kron8(W_e1[l, :16])
        b18 = jnp.tile(b_e1[l], 8).reshape(1, 512)
        k8w2 = _kron8(W_e2[l])
        b28 = jnp.tile(b_e2[l], 8).reshape(1, 128)
        Ga = _sc_gather(pd, dst_a, 1000)
        Gb = _sc_gather(pd, dst_b, 1000)
        ea = _tc_edge(ea, Ga.reshape(eh // 2, 128), ps[:n // 2], ma,
                      k8u, b18, k8w2, b28)
        aggPa = _sc_scatter_add(ea.reshape(eh, 16), dst_a, zinit, 1000)
        eb = _tc_edge(eb, Gb.reshape(eh // 2, 128), ps[n // 2:], mb,
                      k8u, b18, k8w2, b28)
        aggPb = _sc_scatter_add(eb.reshape(eh, 16), dst_b, zinit, 1000)
        ln = (l + 1) % nl
        node, ps, pd = _tc_node(node, aggPa, aggPb, rinv, W_n1[l, :64],
                                W_n1[l, 64:80], b_n1[l].reshape(1, 64),
                                W_n2[l], b_n2[l].reshape(1, 64),
                                W_e1[ln, 16:80], W_e1[ln, 80:144])

    ep = ((b * p + 255) // 256) * 256
    cai = jnp.concatenate([crystal_atom_idx.reshape(b * p).astype(jnp.int32),
                           jnp.zeros((ep - b * p,), jnp.int32)])
    R = _sc_gather(node, cai, ep // 32)
    col = jnp.arange(ep)
    avg = ((col[None, :] // p == jnp.arange(b)[:, None])
           & (col[None, :] < b * p)).astype(_F32) / p
    return _tc_readout(R, avg, W_r, b_r.reshape(1, 128), W_o,
                       b_o.reshape(1, 1))


# speedup vs baseline: 1.0620x; 1.0620x over previous
import functools

import jax
import jax.numpy as jnp
from jax import lax
from jax.experimental import pallas as pl
from jax.experimental.pallas import tpu as pltpu
from jax.experimental.pallas import tpu_sc as plsc

_F32 = jnp.float32
_BF16 = jnp.bfloat16


def _silu(x):
    return x * jax.nn.sigmoid(x)


def _mm(a, b):
    return jnp.dot(a.astype(_BF16), b.astype(_BF16),
                   preferred_element_type=_F32)


def _kron8(w):
    return jnp.kron(jnp.eye(8, dtype=_F32), w)


def _sc_gather(table, idx, chunk):
    n_rows, d = table.shape
    e = idx.shape[0]
    nw = 32
    epw = e // nw
    nch = epw // chunk
    mesh = plsc.VectorSubcoreMesh(core_axis_name="c", subcore_axis_name="s")

    @functools.partial(
        pl.kernel,
        mesh=mesh,
        compiler_params=pltpu.CompilerParams(use_tc_tiling_on_sc=False,
                                             needs_layout_passes=False),
        out_type=jax.ShapeDtypeStruct((e, d), table.dtype),
        scratch_types=[
            pltpu.VMEM((chunk,), jnp.int32),
            pltpu.VMEM((chunk,), jnp.int32),
            pltpu.VMEM((chunk, d), table.dtype),
            pltpu.VMEM((chunk, d), table.dtype),
            pltpu.SemaphoreType.DMA,
            pltpu.SemaphoreType.DMA,
        ],
    )
    def k(table_hbm, idx_hbm, out_hbm, i0, i1, r0, r1, s0, s1):
        wid = lax.axis_index("s") * 2 + lax.axis_index("c")
        base0 = wid * epw
        ib, rb, sb = [i0, i1], [r0, r1], [s0, s1]
        pltpu.sync_copy(idx_hbm.at[pl.ds(base0, chunk)], i0)
        descs = [pltpu.async_copy(table_hbm.at[i0], r0, s0), None]
        for c in range(nch):
            cur, nxt = c % 2, (c + 1) % 2
            if c + 1 < nch:
                pltpu.sync_copy(
                    idx_hbm.at[pl.ds(base0 + (c + 1) * chunk, chunk)], ib[nxt])
                descs[nxt] = pltpu.async_copy(table_hbm.at[ib[nxt]], rb[nxt],
                                              sb[nxt])
            descs[cur].wait()
            pltpu.sync_copy(rb[cur], out_hbm.at[pl.ds(base0 + c * chunk,
                                                      chunk)])

    return k(table, idx)


def _sc_scatter_add(vals, idx, zinit, chunk):
    e = vals.shape[0]
    n_rows = zinit.shape[0]
    nw = 32
    epw = e // nw
    nch = epw // chunk
    rps = n_rows // 16
    mesh = plsc.VectorSubcoreMesh(core_axis_name="c", subcore_axis_name="s")

    @functools.partial(
        pl.kernel,
        mesh=mesh,
        compiler_params=pltpu.CompilerParams(use_tc_tiling_on_sc=False,
                                             needs_layout_passes=False),
        out_type=jax.ShapeDtypeStruct((2, n_rows, 16), _F32),
        scratch_types=[
            pltpu.VMEM((chunk,), jnp.int32),
            pltpu.VMEM((chunk,), jnp.int32),
            pltpu.VMEM((chunk, 16), _F32),
            pltpu.VMEM((chunk, 16), _F32),
            pltpu.SemaphoreType.DMA,
            pltpu.SemaphoreType.DMA,
            pltpu.VMEM_SHARED((n_rows, 16), _F32),
        ],
    )
    def k(vals_hbm, idx_hbm, zinit_hbm, out_hbm, i0, i1, v0, v1, s0, s1,
          shared):
        cid = lax.axis_index("c")
        sid = lax.axis_index("s")
        wid = sid * 2 + cid
        base0 = wid * epw
        ib, vb, sb = [i0, i1], [v0, v1], [s0, s1]
        pltpu.sync_copy(zinit_hbm.at[pl.ds(sid * rps, rps)],
                        v0.at[pl.ds(0, rps)])
        pltpu.sync_copy(v0.at[pl.ds(0, rps)],
                        shared.at[pl.ds(sid * rps, rps)])
        plsc.subcore_barrier()

        pltpu.sync_copy(idx_hbm.at[pl.ds(base0, chunk)], i0)
        descs = [pltpu.async_copy(vals_hbm.at[pl.ds(base0, chunk)], v0, s0),
                 None]
        for c in range(nch):
            cur, nxt = c % 2, (c + 1) % 2
            if c + 1 < nch:
                base_n = base0 + (c + 1) * chunk
                pltpu.sync_copy(idx_hbm.at[pl.ds(base_n, chunk)], ib[nxt])
                descs[nxt] = pltpu.async_copy(
                    vals_hbm.at[pl.ds(base_n, chunk)], vb[nxt], sb[nxt])
            descs[cur].wait()
            pltpu.sync_copy(vb[cur], shared.at[ib[cur]], add=True)

        plsc.subcore_barrier()
        pltpu.sync_copy(shared.at[pl.ds(sid * rps, rps)],
                        v0.at[pl.ds(0, rps)])
        pltpu.sync_copy(v0.at[pl.ds(0, rps)],
                        out_hbm.at[cid, pl.ds(sid * rps, rps)])

    return k(vals, idx, zinit)


def _tc_init(atom_fea, edge_attr_p, W_atom, b_atom, K8We, b_edge8, K8ones,
             Ws0, Wd0):
    n, _ = atom_fea.shape
    e8 = edge_attr_p.shape[0]
    bn = 400
    b8 = bn * 4
    grid = n // bn

    def body(af, ea, wa, ba, we, beb, ko, ws, wd, node_o, ps_o, pd_o, edge_o,
             mf_o):
        nd = _mm(af[...], wa[...]) + ba[...]
        node_o[...] = nd
        ps_o[...] = _mm(nd, ws[...])
        pd_o[...] = _mm(nd, wd[...])
        ea_v = ea[...]
        gsum = _mm(jnp.abs(ea_v), ko[...])
        mf = (gsum > 1e-06).astype(_F32)
        mf_o[...] = mf
        edge_o[...] = (_mm(ea_v, we[...]) + beb[...]) * mf

    return pl.pallas_call(
        body,
        grid=(grid,),
        in_specs=[
            pl.BlockSpec((bn, 128), lambda i: (i, 0)),
            pl.BlockSpec((b8, 128), lambda i: (i, 0)),
            pl.BlockSpec((128, 64), lambda i: (0, 0)),
            pl.BlockSpec((1, 64), lambda i: (0, 0)),
            pl.BlockSpec((128, 128), lambda i: (0, 0)),
            pl.BlockSpec((1, 128), lambda i: (0, 0)),
            pl.BlockSpec((128, 128), lambda i: (0, 0)),
            pl.BlockSpec((64, 64), lambda i: (0, 0)),
            pl.BlockSpec((64, 64), lambda i: (0, 0)),
        ],
        out_specs=[
            pl.BlockSpec((bn, 64), lambda i: (i, 0)),
            pl.BlockSpec((bn, 64), lambda i: (i, 0)),
            pl.BlockSpec((bn, 64), lambda i: (i, 0)),
            pl.BlockSpec((b8, 128), lambda i: (i, 0)),
            pl.BlockSpec((b8, 128), lambda i: (i, 0)),
        ],
        out_shape=[
            jax.ShapeDtypeStruct((n, 64), _F32),
            jax.ShapeDtypeStruct((n, 64), _F32),
            jax.ShapeDtypeStruct((n, 64), _F32),
            jax.ShapeDtypeStruct((e8, 128), _F32),
            jax.ShapeDtypeStruct((e8, 128), _F32),
        ],
    )(atom_fea, edge_attr_p, W_atom, b_atom, K8We, b_edge8, K8ones, Ws0, Wd0)


def _tc_edge(edge_p, G2, ps, mask_p, K8U, b18, K8W2, b28):
    e8 = edge_p.shape[0]
    n = ps.shape[0]
    bn = 1000
    b8 = bn * 4
    b2 = bn * 16
    grid = n // bn

    def body(e_ref, g_ref, ps_ref, mf_ref, u_ref, b1_ref, w2_ref, b2_ref,
             eo_ref):
        psl = jnp.tile(ps_ref[...], (1, 8))
        psb = jnp.broadcast_to(psl[:, None, :], (bn, 4, 512))
        psr = psb.reshape(b8, 512)
        g8 = g_ref[...].reshape(b8, 512)
        ev = e_ref[...]
        pre = _mm(ev, u_ref[...]) + psr + g8 + b1_ref[...]
        h = _silu(pre)
        eo_ref[...] = (ev + _mm(h, w2_ref[...]) + b2_ref[...]) * mf_ref[...]

    return pl.pallas_call(
        body,
        grid=(grid,),
        in_specs=[
            pl.BlockSpec((b8, 128), lambda i: (i, 0)),
            pl.BlockSpec((b2, 128), lambda i: (i, 0)),
            pl.BlockSpec((bn, 64), lambda i: (i, 0)),
            pl.BlockSpec((b8, 128), lambda i: (i, 0)),
            pl.BlockSpec((128, 512), lambda i: (0, 0)),
            pl.BlockSpec((1, 512), lambda i: (0, 0)),
            pl.BlockSpec((512, 128), lambda i: (0, 0)),
            pl.BlockSpec((1, 128), lambda i: (0, 0)),
        ],
        out_specs=pl.BlockSpec((b8, 128), lambda i: (i, 0)),
        out_shape=jax.ShapeDtypeStruct((e8, 128), _F32),
    )(edge_p, G2, ps, mask_p, K8U, b18, K8W2, b28)


def _tc_node(node, aggP, rinv, Wn1a, Wn1b, bn1, Wn2, bn2, Ws, Wd):
    n = node.shape[0]
    bn = n
    grid = n // bn

    def body(nd_ref, ag_ref, ri_ref, w1a, w1b, b1r, w2r, b2r, wsr, wdr,
             no_ref, ps_ref, pd_ref):
        agv = ag_ref[...]
        agg = (agv[0] + agv[1]) * ri_ref[...]
        nd = nd_ref[...]
        h = _silu(_mm(nd, w1a[...]) + _mm(agg, w1b[...]) + b1r[...])
        nn = nd + _mm(h, w2r[...]) + b2r[...]
        no_ref[...] = nn
        ps_ref[...] = _mm(nn, wsr[...])
        pd_ref[...] = _mm(nn, wdr[...])

    return pl.pallas_call(
        body,
        grid=(grid,),
        in_specs=[
            pl.BlockSpec((bn, 64), lambda i: (i, 0)),
            pl.BlockSpec((2, bn, 16), lambda i: (0, i, 0)),
            pl.BlockSpec((bn, 16), lambda i: (i, 0)),
            pl.BlockSpec((64, 64), lambda i: (0, 0)),
            pl.BlockSpec((16, 64), lambda i: (0, 0)),
            pl.BlockSpec((1, 64), lambda i: (0, 0)),
            pl.BlockSpec((64, 64), lambda i: (0, 0)),
            pl.BlockSpec((1, 64), lambda i: (0, 0)),
            pl.BlockSpec((64, 64), lambda i: (0, 0)),
            pl.BlockSpec((64, 64), lambda i: (0, 0)),
        ],
        out_specs=[
            pl.BlockSpec((bn, 64), lambda i: (i, 0)),
            pl.BlockSpec((bn, 64), lambda i: (i, 0)),
            pl.BlockSpec((bn, 64), lambda i: (i, 0)),
        ],
        out_shape=[
            jax.ShapeDtypeStruct((n, 64), _F32),
            jax.ShapeDtypeStruct((n, 64), _F32),
            jax.ShapeDtypeStruct((n, 64), _F32),
        ],
    )(node, aggP, rinv, Wn1a, Wn1b, bn1, Wn2, bn2, Ws, Wd)


def _tc_rinv(cntP):
    _, n, _ = cntP.shape
    bn = 2000
    grid = n // bn

    def body(c_ref, o_ref):
        cv = c_ref[...]
        cnt = cv[0] + cv[1]
        o_ref[...] = 1.0 / jnp.maximum(cnt, 1.0)

    return pl.pallas_call(
        body,
        grid=(grid,),
        in_specs=[pl.BlockSpec((2, bn, 16), lambda i: (0, i, 0))],
        out_specs=pl.BlockSpec((bn, 16), lambda i: (i, 0)),
        out_shape=jax.ShapeDtypeStruct((n, 16), _F32),
    )(cntP)


def _tc_readout(R, A, W_r, b_r, W_o, b_o):
    b = A.shape[0]
    ep = R.shape[0]

    def body(r_ref, a_ref, wr, br, wo, bo, o_ref):
        crys = _mm(a_ref[...], r_ref[...])
        cr = _silu(_mm(crys, wr[...]) + br[...])
        o_ref[...] = _mm(cr, wo[...]) + bo[...]

    return pl.pallas_call(
        body,
        grid=(1,),
        in_specs=[
            pl.BlockSpec((ep, 64), lambda i: (0, 0)),
            pl.BlockSpec((b, ep), lambda i: (0, 0)),
            pl.BlockSpec((64, 128), lambda i: (0, 0)),
            pl.BlockSpec((1, 128), lambda i: (0, 0)),
            pl.BlockSpec((128, 1), lambda i: (0, 0)),
            pl.BlockSpec((1, 1), lambda i: (0, 0)),
        ],
        out_specs=pl.BlockSpec((b, 1), lambda i: (0, 0)),
        out_shape=jax.ShapeDtypeStruct((b, 1), _F32),
    )(R, A, W_r, b_r, W_o, b_o)


def kernel(atom_fea, nbr_fea, nbr_fea_idx, crystal_atom_idx, W_atom, b_atom,
           W_edge, b_edge, W_e1, b_e1, W_e2, b_e2, W_n1, b_n1, W_n2, b_n2,
           W_r, b_r, W_o, b_o):
    n, m = nbr_fea_idx.shape
    e = n * m
    nl = W_e1.shape[0]
    b, p = crystal_atom_idx.shape

    edge_attr_p = nbr_fea.reshape(e // 8, 128)
    dst = jnp.clip(nbr_fea_idx.reshape(e), 0, n - 1).astype(jnp.int32)
    zinit = jnp.zeros((n, 16), _F32)

    node, ps, pd, edge_p, mask_p = _tc_init(
        atom_fea, edge_attr_p, W_atom, b_atom.reshape(1, 64),
        _kron8(W_edge), jnp.tile(b_edge, 8).reshape(1, 128),
        _kron8(jnp.ones((16, 16), _F32)),
        W_e1[0, 16:80], W_e1[0, 80:144])

    cntP = _sc_scatter_add(mask_p.reshape(e, 16), dst, zinit, 2000)
    rinv = _tc_rinv(cntP)

    for l in range(nl):
        G = _sc_gather(pd, dst, 1000)
        edge_p = _tc_edge(
            edge_p, G.reshape(e // 2, 128), ps, mask_p,
            _kron8(W_e1[l, :16]), jnp.tile(b_e1[l], 8).reshape(1, 512),
            _kron8(W_e2[l]), jnp.tile(b_e2[l], 8).reshape(1, 128))
        aggP = _sc_scatter_add(edge_p.reshape(e, 16), dst, zinit, 2000)
        ln = (l + 1) % nl
        node, ps, pd = _tc_node(node, aggP, rinv, W_n1[l, :64], W_n1[l, 64:80],
                                b_n1[l].reshape(1, 64), W_n2[l],
                                b_n2[l].reshape(1, 64),
                                W_e1[ln, 16:80], W_e1[ln, 80:144])

    ep = ((b * p + 255) // 256) * 256
    cai = jnp.concatenate([crystal_atom_idx.reshape(b * p).astype(jnp.int32),
                           jnp.zeros((ep - b * p,), jnp.int32)])
    R = _sc_gather(node, cai, ep // 32)
    col = jnp.arange(ep)
    avg = ((col[None, :] // p == jnp.arange(b)[:, None])
           & (col[None, :] < b * p)).astype(_F32) / p
    return _tc_readout(R, avg, W_r, b_r.reshape(1, 128), W_o,
                       b_o.reshape(1, 1))
